# baseline (speedup 1.0000x reference)
import functools

import jax
import jax.numpy as jnp
from jax import lax
from jax.experimental import pallas as pl
from jax.experimental.pallas import tpu as pltpu
from jax.experimental.pallas import tpu_sc as plsc

_NC, _NS = 2, 16
_NW = _NC * _NS
_B = 8
_CP = 169
_CH = 176
_NG = _CH // 16
_ROWS = _CP * 3
_N13 = _B * 169 * 3
_N26 = _B * 676 * 3
_N52 = _B * 2704 * 3
_NTOT = _N13 + _N26 + _N52
_F32 = jnp.float32


def _decode_chunk(x, out, V, O, b, ci, W, t, aw, ah, hw, row_base):
  start = ci * _CP
  shift = ci % 8
  dma0 = pl.multiple_of(start - shift, 8)
  pltpu.sync_copy(x.at[b, :, pl.ds(dma0, _CH)], V)
  lane = lax.iota(jnp.int32, 16)
  lane3 = lane * 3
  cols = [jnp.full((16,), j, jnp.int32) for j in range(6)]

  def g_body(g, _):
    goff = g * 16
    sl = pl.ds(goff, 16)
    loc = goff + lane
    pos = dma0 + loc
    hh = (pos // W).astype(_F32)
    ww = (pos % W).astype(_F32)
    mask = (loc >= shift) & (loc < shift + _CP)
    for a in range(3):
      c0 = a * 85
      conf_l = V[c0 + 0, sl]
      sx = V[c0 + 1, sl]
      sy = V[c0 + 2, sl]
      tw = V[c0 + 3, sl]
      th = V[c0 + 4, sl]

      def cls_body(k, carry):
        m, mi, kf = carry
        v = V[c0 + 5 + k, sl]
        gt = v > m
        mi = jnp.where(gt, kf, mi)
        m = jnp.maximum(m, v)
        return m, mi, kf + _F32(1.0)

      m0 = jnp.full((16,), -jnp.inf, _F32)
      mi0 = jnp.zeros((16,), _F32)
      _, cls, _ = lax.fori_loop(0, 80, cls_body, (m0, mi0, _F32(0.0)),
                                unroll=8)

      conf = 1.0 / (1.0 + jnp.exp(-conf_l))
      sxs = 1.0 / (1.0 + jnp.exp(-sx))
      sys_ = 1.0 / (1.0 + jnp.exp(-sy))
      cx = (ww + sxs) * t
      cy = (hh + sys_) * t
      bw = aw[a] * jnp.exp(tw)
      bh = ah[a] * jnp.exp(th)
      x1 = cx - bw * 0.5
      y1 = cy - bh * 0.5
      x2 = x1 + bw
      y2 = y1 + bh
      r = lane3 + ((goff - shift) * 3 + a)
      for j, val in enumerate((conf, x1, y1, x2, y2, cls)):
        plsc.store_scatter(O, [r, cols[j]], val, mask=mask)
    return 0

  lax.fori_loop(0, _NG, g_body, 0)
  row0 = row_base + (b * hw + start) * 3
  pltpu.sync_copy(O.at[pl.ds(0, _ROWS), :], out.at[pl.ds(row0, _ROWS), :])


@functools.partial(
    pl.kernel,
    out_type=jax.ShapeDtypeStruct((_NTOT, 8), _F32),
    mesh=plsc.VectorSubcoreMesh(core_axis_name="c", subcore_axis_name="s"),
    compiler_params=pltpu.CompilerParams(use_tc_tiling_on_sc=False,
                                         needs_layout_passes=False),
    scratch_types=[
        pltpu.VMEM((255, _CH), _F32),
        pltpu.VMEM((_CH * 3, 8), _F32),
        pltpu.VMEM((18, 16), _F32),
    ],
)
def _sc_detect(x13, x26, x52, anc, out, V, O, anc_v):
  pltpu.sync_copy(anc, anc_v)
  wid = lax.axis_index("s") * _NC + lax.axis_index("c")

  def anchor_rows(scale):
    aw = [anc_v[scale * 6 + 2 * a] for a in range(3)]
    ah = [anc_v[scale * 6 + 2 * a + 1] for a in range(3)]
    return aw, ah

  aw13, ah13 = anchor_rows(0)
  aw26, ah26 = anchor_rows(1)
  aw52, ah52 = anchor_rows(2)

  def s52_body(i, _):
    u = wid + _NW * i
    _decode_chunk(x52, out, V, O, u // 16, u % 16, 52, 8.0, aw52, ah52,
                  2704, _N13 + _N26)
    return 0

  lax.fori_loop(0, 4, s52_body, 0)

  _decode_chunk(x26, out, V, O, wid // 4, wid % 4, 26, 16.0, aw26, ah26,
                676, _N13)

  @pl.when(wid < 8)
  def _():
    _decode_chunk(x13, out, V, O, wid, jnp.int32(0), 13, 32.0, aw13, ah13,
                  169, 0)


def kernel(output_13, output_26, output_52, anchors_13, anchors_26,
           anchors_52, thresh):
  del thresh
  x13 = jnp.pad(output_13.reshape(_B, 255, 169), ((0, 0), (0, 0), (0, 7)))
  x26 = jnp.pad(output_26.reshape(_B, 255, 676), ((0, 0), (0, 0), (0, 28)))
  x52 = jnp.pad(output_52.reshape(_B, 255, 2704), ((0, 0), (0, 0), (0, 8)))
  anc = jnp.repeat(
      jnp.concatenate([anchors_13.reshape(6), anchors_26.reshape(6),
                       anchors_52.reshape(6)])[:, None], 16, axis=1)
  out = _sc_detect(x13, x26, x52, anc)
  return out[:, :6]


# speedup vs baseline: 2.2227x; 2.2227x over previous
import functools

import jax
import jax.numpy as jnp
from jax import lax
from jax.experimental import pallas as pl
from jax.experimental.pallas import tpu as pltpu
from jax.experimental.pallas import tpu_sc as plsc

_NC, _NS = 2, 16
_NW = _NC * _NS
_B = 8
_CP = 169
_CH = 176
_NG = _CH // 16
_ROWS = _CP * 3
_N13 = _B * 169 * 3
_N26 = _B * 676 * 3
_N52 = _B * 2704 * 3
_NTOT = _N13 + _N26 + _N52
_F32 = jnp.float32


def _decode_chunk(x, out, V, O, b, ci, W, t, aw, ah, hw, row_base):
  start = ci * _CP
  shift = ci % 8
  dma0 = pl.multiple_of(start - shift, 8)
  pltpu.sync_copy(x.at[b, :, pl.ds(dma0, _CH)], V)
  lane = lax.iota(jnp.int32, 16)
  lane3 = lane * 3
  cols = [jnp.full((16,), j, jnp.int32) for j in range(6)]

  def g_body(g, _):
    goff = g * 16
    sl = pl.ds(goff, 16)
    loc = goff + lane
    pos = dma0 + loc
    hh = (pos // W).astype(_F32)
    ww = (pos % W).astype(_F32)
    mask = (loc >= shift) & (loc < shift + _CP)
    for a in range(3):
      c0 = a * 85
      conf_l = V[c0 + 0, sl]
      sx = V[c0 + 1, sl]
      sy = V[c0 + 2, sl]
      tw = V[c0 + 3, sl]
      th = V[c0 + 4, sl]

      def cls_body(k, carry):
        m, mi, kf = carry
        v = V[c0 + 5 + k, sl]
        gt = v > m
        mi = jnp.where(gt, kf, mi)
        m = jnp.maximum(m, v)
        return m, mi, kf + _F32(1.0)

      m0 = jnp.full((16,), -jnp.inf, _F32)
      mi0 = jnp.zeros((16,), _F32)
      _, cls, _ = lax.fori_loop(0, 80, cls_body, (m0, mi0, _F32(0.0)),
                                unroll=8)

      conf = 1.0 / (1.0 + jnp.exp(-conf_l))
      sxs = 1.0 / (1.0 + jnp.exp(-sx))
      sys_ = 1.0 / (1.0 + jnp.exp(-sy))
      cx = (ww + sxs) * t
      cy = (hh + sys_) * t
      bw = aw[a] * jnp.exp(tw)
      bh = ah[a] * jnp.exp(th)
      x1 = cx - bw * 0.5
      y1 = cy - bh * 0.5
      x2 = x1 + bw
      y2 = y1 + bh
      r = lane3 + ((goff - shift) * 3 + a)
      for j, val in enumerate((conf, x1, y1, x2, y2, cls)):
        plsc.store_scatter(O, [r, cols[j]], val, mask=mask)
    return 0

  lax.fori_loop(0, _NG, g_body, 0)
  row0 = row_base + (b * hw + start) * 3
  pltpu.sync_copy(O.at[pl.ds(0, _ROWS), :], out.at[pl.ds(row0, _ROWS), :])


@functools.partial(
    pl.kernel,
    out_type=jax.ShapeDtypeStruct((_NTOT, 8), _F32),
    mesh=plsc.VectorSubcoreMesh(core_axis_name="c", subcore_axis_name="s"),
    compiler_params=pltpu.CompilerParams(use_tc_tiling_on_sc=False,
                                         needs_layout_passes=False),
    scratch_types=[
        pltpu.VMEM((255, _CH), _F32),
        pltpu.VMEM((_CH * 3, 8), _F32),
        pltpu.VMEM((18, 16), _F32),
    ],
)
def _sc_detect(x13, x26, x52, anc, out, V, O, anc_v):
  pltpu.sync_copy(anc, anc_v)
  wid = lax.axis_index("s") * _NC + lax.axis_index("c")

  def anchor_rows(scale):
    aw = [anc_v[scale * 6 + 2 * a] for a in range(3)]
    ah = [anc_v[scale * 6 + 2 * a + 1] for a in range(3)]
    return aw, ah

  aw13, ah13 = anchor_rows(0)
  aw26, ah26 = anchor_rows(1)
  aw52, ah52 = anchor_rows(2)

  def s52_body(i, _):
    u = wid + _NW * i
    _decode_chunk(x52, out, V, O, u // 16, u % 16, 52, 8.0, aw52, ah52,
                  2704, _N13 + _N26)
    return 0

  lax.fori_loop(0, 4, s52_body, 0)

  _decode_chunk(x26, out, V, O, wid // 4, wid % 4, 26, 16.0, aw26, ah26,
                676, _N13)

  @pl.when(wid < 8)
  def _():
    _decode_chunk(x13, out, V, O, wid, jnp.int32(0), 13, 32.0, aw13, ah13,
                  169, 0)


def kernel(output_13, output_26, output_52, anchors_13, anchors_26,
           anchors_52, thresh):
  del thresh
  x13 = jnp.pad(output_13.reshape(_B, 255, 169), ((0, 0), (0, 0), (0, 7)))
  x26 = jnp.pad(output_26.reshape(_B, 255, 676), ((0, 0), (0, 0), (0, 28)))
  x52 = jnp.pad(output_52.reshape(_B, 255, 2704), ((0, 0), (0, 0), (0, 112)))
  anc = jnp.repeat(
      jnp.concatenate([anchors_13.reshape(6), anchors_26.reshape(6),
                       anchors_52.reshape(6)])[:, None], 16, axis=1)
  out = _sc_detect(x13, x26, x52, anc)
  return out[:, :6]


# speedup vs baseline: 2.2683x; 1.0205x over previous
import functools

import jax
import jax.numpy as jnp
from jax import lax
from jax.experimental import pallas as pl
from jax.experimental.pallas import tpu as pltpu
from jax.experimental.pallas import tpu_sc as plsc

_NC, _NS = 2, 16
_NW = _NC * _NS
_B = 8
_CP = 169
_CH = 176
_NG = _CH // 16
_ROWS = _CP * 3
_N13 = _B * 169 * 3
_N26 = _B * 676 * 3
_N52 = _B * 2704 * 3
_NTOT = _N13 + _N26 + _N52
_F32 = jnp.float32


def _decode_chunk(x, out, V, O, b, ci, W, t, aw, ah, hw, row_base):
  start = ci * _CP
  shift = ci % 8
  dma0 = pl.multiple_of(start - shift, 8)
  pltpu.sync_copy(x.at[b, :, pl.ds(dma0, _CH)], V)
  lane = lax.iota(jnp.int32, 16)
  lane3 = lane * 3
  cols = [jnp.full((16,), j, jnp.int32) for j in range(6)]

  def g_body(g, _):
    goff = g * 16
    sl = pl.ds(goff, 16)
    loc = goff + lane
    pos = dma0 + loc
    hh = (pos // W).astype(_F32)
    ww = (pos % W).astype(_F32)
    mask = (loc >= shift) & (loc < shift + _CP)
    for a in range(3):
      c0 = a * 85
      conf_l = V[c0 + 0, sl]
      sx = V[c0 + 1, sl]
      sy = V[c0 + 2, sl]
      tw = V[c0 + 3, sl]
      th = V[c0 + 4, sl]

      def cls_body(k, carry):
        m, mi, kf = carry
        v = V[c0 + 5 + k, sl]
        gt = v > m
        mi = jnp.where(gt, kf, mi)
        m = jnp.maximum(m, v)
        return m, mi, kf + _F32(1.0)

      m0 = jnp.full((16,), -jnp.inf, _F32)
      mi0 = jnp.zeros((16,), _F32)
      _, cls, _ = lax.fori_loop(0, 80, cls_body, (m0, mi0, _F32(0.0)),
                                unroll=8)

      conf = 1.0 / (1.0 + jnp.exp(-conf_l))
      sxs = 1.0 / (1.0 + jnp.exp(-sx))
      sys_ = 1.0 / (1.0 + jnp.exp(-sy))
      cx = (ww + sxs) * t
      cy = (hh + sys_) * t
      bw = aw[a] * jnp.exp(tw)
      bh = ah[a] * jnp.exp(th)
      x1 = cx - bw * 0.5
      y1 = cy - bh * 0.5
      x2 = x1 + bw
      y2 = y1 + bh
      r = lane3 + ((goff - shift) * 3 + a)
      for j, val in enumerate((conf, x1, y1, x2, y2, cls)):
        plsc.store_scatter(O, [r, cols[j]], val, mask=mask)
    return 0

  lax.fori_loop(0, _NG, g_body, 0)
  row0 = row_base + (b * hw + start) * 3
  pltpu.sync_copy(O.at[pl.ds(0, _ROWS), :], out.at[pl.ds(row0, _ROWS), :])


@functools.partial(
    pl.kernel,
    out_type=jax.ShapeDtypeStruct((_NTOT, 8), _F32),
    mesh=plsc.VectorSubcoreMesh(core_axis_name="c", subcore_axis_name="s"),
    compiler_params=pltpu.CompilerParams(use_tc_tiling_on_sc=False,
                                         needs_layout_passes=False),
    scratch_types=[
        pltpu.VMEM((255, _CH), _F32),
        pltpu.VMEM((_CH * 3, 8), _F32),
        pltpu.VMEM((18, 16), _F32),
    ],
)
def _sc_detect(x13, x26, x52, anc, out, V, O, anc_v):
  pltpu.sync_copy(anc, anc_v)
  wid = lax.axis_index("s") * _NC + lax.axis_index("c")

  def anchor_rows(scale):
    aw = [anc_v[scale * 6 + 2 * a] for a in range(3)]
    ah = [anc_v[scale * 6 + 2 * a + 1] for a in range(3)]
    return aw, ah

  aw13, ah13 = anchor_rows(0)
  aw26, ah26 = anchor_rows(1)
  aw52, ah52 = anchor_rows(2)

  def s52_body(i, _):
    u = wid + _NW * i
    _decode_chunk(x52, out, V, O, u // 16, u % 16, 52, 8.0, aw52, ah52,
                  2704, _N13 + _N26)
    return 0

  lax.fori_loop(0, 4, s52_body, 0)

  _decode_chunk(x26, out, V, O, wid // 4, wid % 4, 26, 16.0, aw26, ah26,
                676, _N13)

  @pl.when(wid < 8)
  def _():
    _decode_chunk(x13, out, V, O, wid, jnp.int32(0), 13, 32.0, aw13, ah13,
                  169, 0)


def kernel(output_13, output_26, output_52, anchors_13, anchors_26,
           anchors_52, thresh):
  del thresh
  x13 = jnp.pad(output_13.reshape(_B, 255, 169), ((0, 0), (0, 0), (0, 7)))
  x26 = jnp.pad(output_26.reshape(_B, 255, 676), ((0, 0), (0, 0), (0, 92)))
  x52 = jnp.pad(output_52.reshape(_B, 255, 2704), ((0, 0), (0, 0), (0, 112)))
  anc = jnp.repeat(
      jnp.concatenate([anchors_13.reshape(6), anchors_26.reshape(6),
                       anchors_52.reshape(6)])[:, None], 16, axis=1)
  out = _sc_detect(x13, x26, x52, anc)
  return out[:, :6]


# speedup vs baseline: 2.2739x; 1.0025x over previous
import functools

import jax
import jax.numpy as jnp
from jax import lax
from jax.experimental import pallas as pl
from jax.experimental.pallas import tpu as pltpu
from jax.experimental.pallas import tpu_sc as plsc

_NC, _NS = 2, 16
_NW = _NC * _NS
_B = 8
_CP = 169
_CH = 176
_NG = _CH // 16
_ROWS = _CP * 3
_N13 = _B * 169 * 3
_N26 = _B * 676 * 3
_N52 = _B * 2704 * 3
_NTOT = _N13 + _N26 + _N52
_F32 = jnp.float32


def _decode_chunk(x, out, V, O, b, ci, W, t, aw, ah, hw, row_base):
  start = ci * _CP
  shift = ci % 8
  dma0 = pl.multiple_of(start - shift, 8)
  pltpu.sync_copy(x.at[b, :, pl.ds(dma0, _CH)], V)
  lane = lax.iota(jnp.int32, 16)
  lane3 = lane * 3
  cols = [jnp.full((16,), j, jnp.int32) for j in range(6)]

  def g_body(g, _):
    goff = g * 16
    sl = pl.ds(goff, 16)
    loc = goff + lane
    pos = dma0 + loc
    hh = (pos // W).astype(_F32)
    ww = (pos % W).astype(_F32)
    mask = (loc >= shift) & (loc < shift + _CP)
    for a in range(3):
      c0 = a * 85
      conf_l = V[c0 + 0, sl]
      sx = V[c0 + 1, sl]
      sy = V[c0 + 2, sl]
      tw = V[c0 + 3, sl]
      th = V[c0 + 4, sl]

      def cls_body(k, carry):
        m, mi, kf = carry
        v = V[c0 + 5 + k, sl]
        gt = v > m
        mi = jnp.where(gt, kf, mi)
        m = jnp.maximum(m, v)
        return m, mi, kf + _F32(1.0)

      m0 = jnp.full((16,), -jnp.inf, _F32)
      mi0 = jnp.zeros((16,), _F32)
      _, cls, _ = lax.fori_loop(0, 80, cls_body, (m0, mi0, _F32(0.0)),
                                unroll=8)

      conf = 1.0 / (1.0 + jnp.exp(-conf_l))
      sxs = 1.0 / (1.0 + jnp.exp(-sx))
      sys_ = 1.0 / (1.0 + jnp.exp(-sy))
      cx = (ww + sxs) * t
      cy = (hh + sys_) * t
      bw = aw[a] * jnp.exp(tw)
      bh = ah[a] * jnp.exp(th)
      x1 = cx - bw * 0.5
      y1 = cy - bh * 0.5
      x2 = x1 + bw
      y2 = y1 + bh
      r = lane3 + ((goff - shift) * 3 + a)
      for j, val in enumerate((conf, x1, y1, x2, y2, cls)):
        plsc.store_scatter(O, [r, cols[j]], val, mask=mask)
    return 0

  lax.fori_loop(0, _NG, g_body, 0)
  row0 = row_base + (b * hw + start) * 3
  pltpu.sync_copy(O.at[pl.ds(0, _ROWS), :], out.at[pl.ds(row0, _ROWS), :])


@functools.partial(
    pl.kernel,
    out_type=jax.ShapeDtypeStruct((_NTOT, 8), _F32),
    mesh=plsc.VectorSubcoreMesh(core_axis_name="c", subcore_axis_name="s"),
    compiler_params=pltpu.CompilerParams(use_tc_tiling_on_sc=False,
                                         needs_layout_passes=False),
    scratch_types=[
        pltpu.VMEM((255, _CH), _F32),
        pltpu.VMEM((_CH * 3, 8), _F32),
        pltpu.VMEM((18, 16), _F32),
    ],
)
def _sc_detect(x13, x26, x52, anc, out, V, O, anc_v):
  pltpu.sync_copy(anc, anc_v)
  wid = lax.axis_index("s") * _NC + lax.axis_index("c")

  def anchor_rows(scale):
    aw = [anc_v[scale * 6 + 2 * a] for a in range(3)]
    ah = [anc_v[scale * 6 + 2 * a + 1] for a in range(3)]
    return aw, ah

  aw13, ah13 = anchor_rows(0)
  aw26, ah26 = anchor_rows(1)
  aw52, ah52 = anchor_rows(2)

  def s52_body(i, _):
    u = wid + _NW * i
    _decode_chunk(x52, out, V, O, u // 16, u % 16, 52, 8.0, aw52, ah52,
                  2704, _N13 + _N26)
    return 0

  lax.fori_loop(0, 4, s52_body, 0)

  _decode_chunk(x26, out, V, O, wid // 4, wid % 4, 26, 16.0, aw26, ah26,
                676, _N13)

  @pl.when(wid < 8)
  def _():
    _decode_chunk(x13, out, V, O, wid, jnp.int32(0), 13, 32.0, aw13, ah13,
                  169, 0)


def kernel(output_13, output_26, output_52, anchors_13, anchors_26,
           anchors_52, thresh):
  del thresh
  x13 = jnp.pad(output_13.reshape(_B, 255, 169), ((0, 0), (0, 0), (0, 87)))
  x26 = jnp.pad(output_26.reshape(_B, 255, 676), ((0, 0), (0, 0), (0, 92)))
  x52 = jnp.pad(output_52.reshape(_B, 255, 2704), ((0, 0), (0, 0), (0, 112)))
  anc = jnp.repeat(
      jnp.concatenate([anchors_13.reshape(6), anchors_26.reshape(6),
                       anchors_52.reshape(6)])[:, None], 16, axis=1)
  out = _sc_detect(x13, x26, x52, anc)
  return out[:, :6]
